# Initial kernel scaffold; baseline (speedup 1.0000x reference)
import jax
import jax.numpy as jnp
from jax.experimental import pallas as pl


def kernel(x, edge_index, W1, b1, W2, b2):
    raise NotImplementedError("write your pallas kernel here")


# speedup vs baseline: 1.0906x; 1.0906x over previous
import jax
import jax.numpy as jnp
from jax.experimental import pallas as pl


def _copy_kernel(x_ref, o_ref):
    o_ref[...] = x_ref[...]


def kernel(x, edge_index, W1, b1, W2, b2):
    N, D = x.shape
    src = edge_index[0]
    dst = edge_index[1]
    W1a, W1b = W1[:D], W1[D:]
    A = x @ (W1a - W1b) + b1
    B = x @ W1b
    h1 = A[dst] + B[src]
    h1 = jnp.maximum(h1, 0.01 * h1)
    h2 = h1 @ W2 + b2
    h2 = jnp.maximum(h2, 0.01 * h2)
    out = jax.ops.segment_sum(h2, dst, num_segments=N)
    return pl.pallas_call(
        _copy_kernel,
        out_shape=jax.ShapeDtypeStruct(out.shape, out.dtype),
    )(out)


# speedup vs baseline: 3.0572x; 2.8033x over previous
import functools

import jax
import jax.numpy as jnp
from jax import lax
from jax.experimental import pallas as pl
from jax.experimental.pallas import tpu as pltpu
from jax.experimental.pallas import tpu_sc as plsc

NC, NS, LANES = 2, 16, 16
NW = NC * NS
CHUNK = 80


def _node_mm_body(x_ref, w1_ref, b1_ref, a_ref, b_ref):
    xb = x_ref[...]
    d = x_ref.shape[1]
    w1a = w1_ref[:d, :]
    w1b = w1_ref[d:, :]
    a_ref[...] = jnp.dot(xb, w1a - w1b, preferred_element_type=jnp.float32) + b1_ref[...]
    b_ref[...] = jnp.dot(xb, w1b, preferred_element_type=jnp.float32)


def _node_mm(x, W1, b1, block_n=1000):
    n, d = x.shape
    grid = n // block_n
    return pl.pallas_call(
        _node_mm_body,
        grid=(grid,),
        in_specs=[
            pl.BlockSpec((block_n, d), lambda i: (i, 0)),
            pl.BlockSpec((2 * d, d), lambda i: (0, 0)),
            pl.BlockSpec((1, d), lambda i: (0, 0)),
        ],
        out_specs=[
            pl.BlockSpec((block_n, d), lambda i: (i, 0)),
            pl.BlockSpec((block_n, d), lambda i: (i, 0)),
        ],
        out_shape=[
            jax.ShapeDtypeStruct((n, d), jnp.float32),
            jax.ShapeDtypeStruct((n, d), jnp.float32),
        ],
    )(x, W1, b1)


def _gather_body(a_hbm, b_hbm, dst_hbm, src_hbm, h1_hbm,
                 dstv, srcv, arows, brows, hrows, sem1, sem2):
    e = h1_hbm.shape[0]
    per_w = e // NW
    n_chunks = per_w // CHUNK
    c = lax.axis_index("c")
    s = lax.axis_index("s")
    wid = s * NC + c

    def chunk_body(i, carry):
        base = wid * per_w + i * CHUNK
        pltpu.sync_copy(dst_hbm.at[pl.ds(base, CHUNK)], dstv)
        pltpu.sync_copy(src_hbm.at[pl.ds(base, CHUNK)], srcv)
        cp1 = pltpu.async_copy(a_hbm.at[dstv], arows, sem1)
        cp2 = pltpu.async_copy(b_hbm.at[srcv], brows, sem2)
        cp1.wait()
        cp2.wait()

        def row_body(r, rc):
            for g in range(128 // LANES):
                v = arows[r, pl.ds(g * LANES, LANES)] + brows[r, pl.ds(g * LANES, LANES)]
                hrows[r, pl.ds(g * LANES, LANES)] = jnp.maximum(v, 0.01 * v)
            return rc

        lax.fori_loop(0, CHUNK, row_body, 0)
        pltpu.sync_copy(hrows, h1_hbm.at[pl.ds(base, CHUNK)])
        return carry

    lax.fori_loop(0, n_chunks, chunk_body, 0)


def _edge_gather(A, B, dst, src, E):
    d = A.shape[1]
    mesh = plsc.VectorSubcoreMesh(core_axis_name="c", subcore_axis_name="s")
    return pl.kernel(
        _gather_body,
        out_type=jax.ShapeDtypeStruct((E, d), jnp.float32),
        mesh=mesh,
        scratch_types=[
            pltpu.VMEM((CHUNK,), jnp.int32),
            pltpu.VMEM((CHUNK,), jnp.int32),
            pltpu.VMEM((CHUNK, d), jnp.float32),
            pltpu.VMEM((CHUNK, d), jnp.float32),
            pltpu.VMEM((CHUNK, d), jnp.float32),
            pltpu.SemaphoreType.DMA,
            pltpu.SemaphoreType.DMA,
        ],
    )(A, B, dst, src)


def _edge_mm_body(h1_ref, w2_ref, b2_ref, h2_ref):
    h = jnp.dot(h1_ref[...], w2_ref[...], preferred_element_type=jnp.float32) + b2_ref[...]
    h2_ref[...] = jnp.maximum(h, 0.01 * h)


def _edge_mm(h1, W2, b2, block_e=2000):
    e, d = h1.shape
    grid = e // block_e
    return pl.pallas_call(
        _edge_mm_body,
        grid=(grid,),
        in_specs=[
            pl.BlockSpec((block_e, d), lambda i: (i, 0)),
            pl.BlockSpec((d, d), lambda i: (0, 0)),
            pl.BlockSpec((1, d), lambda i: (0, 0)),
        ],
        out_specs=pl.BlockSpec((block_e, d), lambda i: (i, 0)),
        out_shape=jax.ShapeDtypeStruct((e, d), jnp.float32),
    )(h1, W2, b2)


def _scatter_body(h2_hbm, dst_hbm, zeros_hbm, parts_hbm, dstv, rows, acc, sem):
    e, d = h2_hbm.shape
    n = zeros_hbm.shape[0]
    per_w = e // NW
    n_chunks = per_w // CHUNK
    stripe = (n // NS) // 8 * 8
    tail = n - NS * stripe
    c = lax.axis_index("c")
    s = lax.axis_index("s")
    wid = s * NC + c

    pltpu.sync_copy(zeros_hbm.at[pl.ds(s * stripe, stripe)],
                    acc.at[pl.ds(s * stripe, stripe)])
    if tail:
        @pl.when(s == 0)
        def _():
            pltpu.sync_copy(zeros_hbm.at[pl.ds(NS * stripe, tail)],
                            acc.at[pl.ds(NS * stripe, tail)])
    plsc.subcore_barrier()

    def chunk_body(i, carry):
        base = wid * per_w + i * CHUNK
        pltpu.sync_copy(dst_hbm.at[pl.ds(base, CHUNK)], dstv)
        pltpu.sync_copy(h2_hbm.at[pl.ds(base, CHUNK)], rows)
        pltpu.sync_copy(rows, acc.at[dstv], add=True)
        return carry

    lax.fori_loop(0, n_chunks, chunk_body, 0)
    plsc.subcore_barrier()
    pltpu.sync_copy(acc.at[pl.ds(s * stripe, stripe)],
                    parts_hbm.at[c, pl.ds(s * stripe, stripe)])
    if tail:
        @pl.when(s == 0)
        def _():
            pltpu.sync_copy(acc.at[pl.ds(NS * stripe, tail)],
                            parts_hbm.at[c, pl.ds(NS * stripe, tail)])


def _edge_scatter(h2, dst, zeros, N):
    e, d = h2.shape
    mesh = plsc.VectorSubcoreMesh(core_axis_name="c", subcore_axis_name="s")
    return pl.kernel(
        _scatter_body,
        out_type=jax.ShapeDtypeStruct((NC, N, d), jnp.float32),
        mesh=mesh,
        scratch_types=[
            pltpu.VMEM((CHUNK,), jnp.int32),
            pltpu.VMEM((CHUNK, d), jnp.float32),
            pltpu.VMEM_SHARED((N, d), jnp.float32),
            pltpu.SemaphoreType.DMA,
        ],
    )(h2, dst, zeros)


def _combine_body(p0_ref, p1_ref, o_ref):
    o_ref[...] = p0_ref[...] + p1_ref[...]


def _combine(parts, block_n=1000):
    _, n, d = parts.shape
    grid = n // block_n
    return pl.pallas_call(
        _combine_body,
        grid=(grid,),
        in_specs=[
            pl.BlockSpec((block_n, d), lambda i: (i, 0)),
            pl.BlockSpec((block_n, d), lambda i: (i, 0)),
        ],
        out_specs=pl.BlockSpec((block_n, d), lambda i: (i, 0)),
        out_shape=jax.ShapeDtypeStruct((n, d), jnp.float32),
    )(parts[0], parts[1])


def kernel(x, edge_index, W1, b1, W2, b2):
    n, d = x.shape
    e = edge_index.shape[1]
    src = edge_index[0]
    dst = edge_index[1]
    A, B = _node_mm(x, W1, b1.reshape(1, d))
    h1 = _edge_gather(A, B, dst, src, e)
    h2 = _edge_mm(h1, W2, b2.reshape(1, d))
    zeros = jnp.zeros((n, d), jnp.float32)
    parts = _edge_scatter(h2, dst, zeros, n)
    return _combine(parts)


# speedup vs baseline: 5.8418x; 1.9108x over previous
import functools

import jax
import jax.numpy as jnp
from jax import lax
from jax.experimental import pallas as pl
from jax.experimental.pallas import tpu as pltpu
from jax.experimental.pallas import tpu_sc as plsc

NC, NS, LANES = 2, 16, 16
NW = NC * NS
CHUNK = 80


def _node_mm_body(x_ref, w1_ref, b1_ref, a_ref, b_ref):
    xb = x_ref[...]
    d = x_ref.shape[1]
    w1a = w1_ref[:d, :]
    w1b = w1_ref[d:, :]
    a_ref[...] = jnp.dot(xb, w1a - w1b, preferred_element_type=jnp.float32) + b1_ref[...]
    b_ref[...] = jnp.dot(xb, w1b, preferred_element_type=jnp.float32)


def _node_mm(x, W1, b1, block_n=1000):
    n, d = x.shape
    grid = n // block_n
    return pl.pallas_call(
        _node_mm_body,
        grid=(grid,),
        in_specs=[
            pl.BlockSpec((block_n, d), lambda i: (i, 0)),
            pl.BlockSpec((2 * d, d), lambda i: (0, 0)),
            pl.BlockSpec((1, d), lambda i: (0, 0)),
        ],
        out_specs=[
            pl.BlockSpec((block_n, d), lambda i: (i, 0)),
            pl.BlockSpec((block_n, d), lambda i: (i, 0)),
        ],
        out_shape=[
            jax.ShapeDtypeStruct((n, d), jnp.float32),
            jax.ShapeDtypeStruct((n, d), jnp.float32),
        ],
    )(x, W1, b1)


def _gather_body(a_hbm, b_hbm, dst_hbm, src_hbm, h1_hbm,
                 dsti, srci, arows, brows, hrows,
                 gsem0, gsem1, wsem0, wsem1):
    e, d = h1_hbm.shape
    per_w = e // NW
    n_chunks = per_w // CHUNK
    assert n_chunks % 2 == 1
    pairs = (n_chunks - 1) // 2
    c_ax = lax.axis_index("c")
    s_ax = lax.axis_index("s")
    wid = s_ax * NC + c_ax
    base_w = wid * per_w
    gsems = (gsem0, gsem1)
    wsems = (wsem0, wsem1)

    pltpu.sync_copy(dst_hbm.at[pl.ds(base_w, per_w)], dsti)
    pltpu.sync_copy(src_hbm.at[pl.ds(base_w, per_w)], srci)

    def issue_gather(ci, b):
        i0 = ci * CHUNK
        pltpu.async_copy(a_hbm.at[dsti.at[pl.ds(i0, CHUNK)]], arows.at[b], gsems[b])
        pltpu.async_copy(b_hbm.at[srci.at[pl.ds(i0, CHUNK)]], brows.at[b], gsems[b])

    def wait_gather(b):
        pltpu.make_async_copy(a_hbm.at[pl.ds(0, CHUNK)], arows.at[b], gsems[b]).wait()
        pltpu.make_async_copy(b_hbm.at[pl.ds(0, CHUNK)], brows.at[b], gsems[b]).wait()

    def wait_wb(b):
        pltpu.make_async_copy(hrows.at[b], h1_hbm.at[pl.ds(0, CHUNK)], wsems[b]).wait()

    def compute(b):
        ar = arows.at[b]
        br = brows.at[b]
        hr = hrows.at[b]

        def row_body(r, rc):
            r4 = r * 4
            for rr in range(4):
                for g in range(d // LANES):
                    sl = pl.ds(g * LANES, LANES)
                    v = ar[r4 + rr, sl] + br[r4 + rr, sl]
                    hr[r4 + rr, sl] = jnp.maximum(v, 0.01 * v)
            return rc

        lax.fori_loop(0, CHUNK // 4, row_body, 0)

    issue_gather(0, 0)
    issue_gather(1, 1)

    def pair_body(p, carry):
        for b in range(2):
            c = p * 2 + b
            wait_gather(b)

            @pl.when(p >= 1)
            def _():
                wait_wb(b)

            compute(b)
            pltpu.async_copy(hrows.at[b], h1_hbm.at[pl.ds(base_w + c * CHUNK, CHUNK)],
                             wsems[b])
            if b == 0:
                issue_gather(c + 2, b)
            else:
                @pl.when(p < pairs - 1)
                def _():
                    issue_gather(c + 2, b)
        return carry

    lax.fori_loop(0, pairs, pair_body, 0)

    c = n_chunks - 1
    wait_gather(0)
    wait_wb(0)
    compute(0)
    pltpu.sync_copy(hrows.at[0], h1_hbm.at[pl.ds(base_w + c * CHUNK, CHUNK)])
    wait_wb(1)


def _edge_gather(A, B, dst, src, E):
    d = A.shape[1]
    per_w = E // NW
    mesh = plsc.VectorSubcoreMesh(core_axis_name="c", subcore_axis_name="s")
    return pl.kernel(
        _gather_body,
        out_type=jax.ShapeDtypeStruct((E, d), jnp.float32),
        mesh=mesh,
        scratch_types=[
            pltpu.VMEM((per_w,), jnp.int32),
            pltpu.VMEM((per_w,), jnp.int32),
            pltpu.VMEM((2, CHUNK, d), jnp.float32),
            pltpu.VMEM((2, CHUNK, d), jnp.float32),
            pltpu.VMEM((2, CHUNK, d), jnp.float32),
            pltpu.SemaphoreType.DMA,
            pltpu.SemaphoreType.DMA,
            pltpu.SemaphoreType.DMA,
            pltpu.SemaphoreType.DMA,
        ],
    )(A, B, dst, src)


def _edge_mm_body(h1_ref, w2_ref, b2_ref, h2_ref):
    h = jnp.dot(h1_ref[...], w2_ref[...], preferred_element_type=jnp.float32) + b2_ref[...]
    h2_ref[...] = jnp.maximum(h, 0.01 * h)


def _edge_mm(h1, W2, b2, block_e=4000):
    e, d = h1.shape
    grid = e // block_e
    return pl.pallas_call(
        _edge_mm_body,
        grid=(grid,),
        in_specs=[
            pl.BlockSpec((block_e, d), lambda i: (i, 0)),
            pl.BlockSpec((d, d), lambda i: (0, 0)),
            pl.BlockSpec((1, d), lambda i: (0, 0)),
        ],
        out_specs=pl.BlockSpec((block_e, d), lambda i: (i, 0)),
        out_shape=jax.ShapeDtypeStruct((e, d), jnp.float32),
    )(h1, W2, b2)


def _scatter_body(h2_hbm, dst_hbm, zeros_hbm, parts_hbm,
                  dstv0, dstv1, rows, acc, lsem0, lsem1):
    e, d = h2_hbm.shape
    n = zeros_hbm.shape[0]
    per_w = e // NW
    n_chunks = per_w // CHUNK
    assert n_chunks % 2 == 1
    pairs = (n_chunks - 1) // 2
    stripe = (n // NS) // 8 * 8
    tail = n - NS * stripe
    c_ax = lax.axis_index("c")
    s_ax = lax.axis_index("s")
    wid = s_ax * NC + c_ax
    base_w = wid * per_w
    dstvs = (dstv0, dstv1)
    lsems = (lsem0, lsem1)

    pltpu.sync_copy(zeros_hbm.at[pl.ds(s_ax * stripe, stripe)],
                    acc.at[pl.ds(s_ax * stripe, stripe)])
    if tail:
        @pl.when(s_ax == 0)
        def _():
            pltpu.sync_copy(zeros_hbm.at[pl.ds(NS * stripe, tail)],
                            acc.at[pl.ds(NS * stripe, tail)])
    plsc.subcore_barrier()

    def issue_load(ci, b):
        base = base_w + ci * CHUNK
        pltpu.async_copy(dst_hbm.at[pl.ds(base, CHUNK)], dstvs[b], lsems[b])
        pltpu.async_copy(h2_hbm.at[pl.ds(base, CHUNK)], rows.at[b], lsems[b])

    def wait_load(b):
        pltpu.make_async_copy(dst_hbm.at[pl.ds(0, CHUNK)], dstvs[b], lsems[b]).wait()
        pltpu.make_async_copy(h2_hbm.at[pl.ds(0, CHUNK)], rows.at[b], lsems[b]).wait()

    issue_load(0, 0)
    issue_load(1, 1)

    def pair_body(p, carry):
        for b in range(2):
            c = p * 2 + b
            wait_load(b)
            pltpu.sync_copy(rows.at[b], acc.at[dstvs[b]], add=True)
            if b == 0:
                issue_load(c + 2, b)
            else:
                @pl.when(p < pairs - 1)
                def _():
                    issue_load(c + 2, b)
        return carry

    lax.fori_loop(0, pairs, pair_body, 0)
    wait_load(0)
    pltpu.sync_copy(rows.at[0], acc.at[dstv0], add=True)

    plsc.subcore_barrier()
    pltpu.sync_copy(acc.at[pl.ds(s_ax * stripe, stripe)],
                    parts_hbm.at[c_ax, pl.ds(s_ax * stripe, stripe)])
    if tail:
        @pl.when(s_ax == 0)
        def _():
            pltpu.sync_copy(acc.at[pl.ds(NS * stripe, tail)],
                            parts_hbm.at[c_ax, pl.ds(NS * stripe, tail)])


def _edge_scatter(h2, dst, zeros, N):
    e, d = h2.shape
    mesh = plsc.VectorSubcoreMesh(core_axis_name="c", subcore_axis_name="s")
    return pl.kernel(
        _scatter_body,
        out_type=jax.ShapeDtypeStruct((NC, N, d), jnp.float32),
        mesh=mesh,
        scratch_types=[
            pltpu.VMEM((CHUNK,), jnp.int32),
            pltpu.VMEM((CHUNK,), jnp.int32),
            pltpu.VMEM((2, CHUNK, d), jnp.float32),
            pltpu.VMEM_SHARED((N, d), jnp.float32),
            pltpu.SemaphoreType.DMA,
            pltpu.SemaphoreType.DMA,
        ],
    )(h2, dst, zeros)


def _combine_body(p0_ref, p1_ref, o_ref):
    o_ref[...] = p0_ref[...] + p1_ref[...]


def _combine(parts, block_n=1000):
    _, n, d = parts.shape
    grid = n // block_n
    return pl.pallas_call(
        _combine_body,
        grid=(grid,),
        in_specs=[
            pl.BlockSpec((block_n, d), lambda i: (i, 0)),
            pl.BlockSpec((block_n, d), lambda i: (i, 0)),
        ],
        out_specs=pl.BlockSpec((block_n, d), lambda i: (i, 0)),
        out_shape=jax.ShapeDtypeStruct((n, d), jnp.float32),
    )(parts[0], parts[1])


def kernel(x, edge_index, W1, b1, W2, b2):
    n, d = x.shape
    e = edge_index.shape[1]
    src = edge_index[0]
    dst = edge_index[1]
    A, B = _node_mm(x, W1, b1.reshape(1, d))
    h1 = _edge_gather(A, B, dst, src, e)
    h2 = _edge_mm(h1, W2, b2.reshape(1, d))
    zeros = jnp.zeros((n, d), jnp.float32)
    parts = _edge_scatter(h2, dst, zeros, n)
    return _combine(parts)


# speedup vs baseline: 5.8660x; 1.0042x over previous
import functools

import jax
import jax.numpy as jnp
from jax import lax
from jax.experimental import pallas as pl
from jax.experimental.pallas import tpu as pltpu
from jax.experimental.pallas import tpu_sc as plsc

NC, NS, LANES = 2, 16, 16
NW = NC * NS
CHUNK = 80


def _node_mm_body(x_ref, w1_ref, b1_ref, a_ref, b_ref):
    xb = x_ref[...]
    d = x_ref.shape[1]
    w1a = w1_ref[:d, :]
    w1b = w1_ref[d:, :]
    a_ref[...] = jnp.dot(xb, w1a - w1b, preferred_element_type=jnp.float32) + b1_ref[...]
    b_ref[...] = jnp.dot(xb, w1b, preferred_element_type=jnp.float32)


def _node_mm(x, W1, b1, block_n=1000):
    n, d = x.shape
    grid = n // block_n
    return pl.pallas_call(
        _node_mm_body,
        grid=(grid,),
        in_specs=[
            pl.BlockSpec((block_n, d), lambda i: (i, 0)),
            pl.BlockSpec((2 * d, d), lambda i: (0, 0)),
            pl.BlockSpec((1, d), lambda i: (0, 0)),
        ],
        out_specs=[
            pl.BlockSpec((block_n, d), lambda i: (i, 0)),
            pl.BlockSpec((block_n, d), lambda i: (i, 0)),
        ],
        out_shape=[
            jax.ShapeDtypeStruct((n, d), jnp.float32),
            jax.ShapeDtypeStruct((n, d), jnp.float32),
        ],
    )(x, W1, b1)


def _gather_body(a_hbm, b_hbm, dst_hbm, src_hbm, h1_hbm,
                 dsti, srci, arows, brows, hrows,
                 gsem0, gsem1, wsem0, wsem1):
    e, d = h1_hbm.shape
    per_w = e // NW
    n_chunks = per_w // CHUNK
    assert n_chunks % 2 == 1
    pairs = (n_chunks - 1) // 2
    c_ax = lax.axis_index("c")
    s_ax = lax.axis_index("s")
    wid = s_ax * NC + c_ax
    base_w = wid * per_w
    gsems = (gsem0, gsem1)
    wsems = (wsem0, wsem1)

    pltpu.sync_copy(dst_hbm.at[pl.ds(base_w, per_w)], dsti)
    pltpu.sync_copy(src_hbm.at[pl.ds(base_w, per_w)], srci)

    def issue_gather(ci, b):
        i0 = ci * CHUNK
        pltpu.async_copy(a_hbm.at[dsti.at[pl.ds(i0, CHUNK)]], arows.at[b], gsems[b])
        pltpu.async_copy(b_hbm.at[srci.at[pl.ds(i0, CHUNK)]], brows.at[b], gsems[b])

    def wait_gather(b):
        pltpu.make_async_copy(a_hbm.at[pl.ds(0, CHUNK)], arows.at[b], gsems[b]).wait()
        pltpu.make_async_copy(b_hbm.at[pl.ds(0, CHUNK)], brows.at[b], gsems[b]).wait()

    def wait_wb(b):
        pltpu.make_async_copy(hrows.at[b], h1_hbm.at[pl.ds(0, CHUNK)], wsems[b]).wait()

    def compute(b):
        ar = arows.at[b]
        br = brows.at[b]
        hr = hrows.at[b]

        def row_body(r, rc):
            r8 = r * 8
            for rr in range(8):
                for g in range(d // LANES):
                    sl = pl.ds(g * LANES, LANES)
                    v = ar[r8 + rr, sl] + br[r8 + rr, sl]
                    hr[r8 + rr, sl] = jnp.maximum(v, 0.01 * v)
            return rc

        lax.fori_loop(0, CHUNK // 8, row_body, 0)

    issue_gather(0, 0)
    issue_gather(1, 1)

    def pair_body(p, carry):
        for b in range(2):
            c = p * 2 + b
            wait_gather(b)

            @pl.when(p >= 1)
            def _():
                wait_wb(b)

            compute(b)
            pltpu.async_copy(hrows.at[b], h1_hbm.at[pl.ds(base_w + c * CHUNK, CHUNK)],
                             wsems[b])
            if b == 0:
                issue_gather(c + 2, b)
            else:
                @pl.when(p < pairs - 1)
                def _():
                    issue_gather(c + 2, b)
        return carry

    lax.fori_loop(0, pairs, pair_body, 0)

    c = n_chunks - 1
    wait_gather(0)
    wait_wb(0)
    compute(0)
    pltpu.sync_copy(hrows.at[0], h1_hbm.at[pl.ds(base_w + c * CHUNK, CHUNK)])
    wait_wb(1)


def _edge_gather(A, B, dst, src, E):
    d = A.shape[1]
    per_w = E // NW
    mesh = plsc.VectorSubcoreMesh(core_axis_name="c", subcore_axis_name="s")
    return pl.kernel(
        _gather_body,
        out_type=jax.ShapeDtypeStruct((E, d), jnp.float32),
        mesh=mesh,
        scratch_types=[
            pltpu.VMEM((per_w,), jnp.int32),
            pltpu.VMEM((per_w,), jnp.int32),
            pltpu.VMEM((2, CHUNK, d), jnp.float32),
            pltpu.VMEM((2, CHUNK, d), jnp.float32),
            pltpu.VMEM((2, CHUNK, d), jnp.float32),
            pltpu.SemaphoreType.DMA,
            pltpu.SemaphoreType.DMA,
            pltpu.SemaphoreType.DMA,
            pltpu.SemaphoreType.DMA,
        ],
    )(A, B, dst, src)


def _edge_mm_body(h1_ref, w2_ref, b2_ref, h2_ref):
    h = jnp.dot(h1_ref[...], w2_ref[...], preferred_element_type=jnp.float32) + b2_ref[...]
    h2_ref[...] = jnp.maximum(h, 0.01 * h)


def _edge_mm(h1, W2, b2, block_e=4000):
    e, d = h1.shape
    grid = e // block_e
    return pl.pallas_call(
        _edge_mm_body,
        grid=(grid,),
        in_specs=[
            pl.BlockSpec((block_e, d), lambda i: (i, 0)),
            pl.BlockSpec((d, d), lambda i: (0, 0)),
            pl.BlockSpec((1, d), lambda i: (0, 0)),
        ],
        out_specs=pl.BlockSpec((block_e, d), lambda i: (i, 0)),
        out_shape=jax.ShapeDtypeStruct((e, d), jnp.float32),
    )(h1, W2, b2)


def _scatter_body(h2_hbm, dst_hbm, zeros_hbm, parts_hbm,
                  dstv0, dstv1, dstv2, dstv3, rows, acc,
                  lsem0, lsem1, lsem2, lsem3,
                  ssem0, ssem1, ssem2, ssem3):
    e, d = h2_hbm.shape
    n = zeros_hbm.shape[0]
    per_w = e // NW
    n_chunks = per_w // CHUNK
    assert n_chunks % 4 == 1
    quads = n_chunks // 4
    stripe = (n // NS) // 8 * 8
    tail = n - NS * stripe
    c_ax = lax.axis_index("c")
    s_ax = lax.axis_index("s")
    wid = s_ax * NC + c_ax
    base_w = wid * per_w
    dstvs = (dstv0, dstv1, dstv2, dstv3)
    lsems = (lsem0, lsem1, lsem2, lsem3)
    ssems = (ssem0, ssem1, ssem2, ssem3)

    pltpu.sync_copy(zeros_hbm.at[pl.ds(s_ax * stripe, stripe)],
                    acc.at[pl.ds(s_ax * stripe, stripe)])
    if tail:
        @pl.when(s_ax == 0)
        def _():
            pltpu.sync_copy(zeros_hbm.at[pl.ds(NS * stripe, tail)],
                            acc.at[pl.ds(NS * stripe, tail)])
    plsc.subcore_barrier()

    def issue_load(ci, b):
        base = base_w + ci * CHUNK
        pltpu.async_copy(dst_hbm.at[pl.ds(base, CHUNK)], dstvs[b], lsems[b])
        pltpu.async_copy(h2_hbm.at[pl.ds(base, CHUNK)], rows.at[b], lsems[b])

    def wait_load(b):
        pltpu.make_async_copy(dst_hbm.at[pl.ds(0, CHUNK)], dstvs[b], lsems[b]).wait()
        pltpu.make_async_copy(h2_hbm.at[pl.ds(0, CHUNK)], rows.at[b], lsems[b]).wait()

    def scatter_go(b):
        pltpu.async_copy(rows.at[b], acc.at[dstvs[b]], ssems[b], add=True)

    def wait_scatter(b):
        pltpu.make_async_copy(h2_hbm.at[pl.ds(0, CHUNK)], rows.at[b], ssems[b]).wait()

    issue_load(0, 0)
    issue_load(1, 1)

    def quad_body(q, carry):
        for i in range(4):
            v = q * 4 + i
            wait_load(i)
            scatter_go(i)
            bw = (i + 2) % 4
            if i < 2:
                @pl.when(q >= 1)
                def _():
                    wait_scatter(bw)

                issue_load(v + 2, bw)
            else:
                @pl.when(v + 2 < n_chunks)
                def _():
                    wait_scatter(bw)
                    issue_load(v + 2, bw)
        return carry

    lax.fori_loop(0, quads, quad_body, 0)
    wait_load(0)
    scatter_go(0)
    for b in range(4):
        wait_scatter(b)

    plsc.subcore_barrier()
    pltpu.sync_copy(acc.at[pl.ds(s_ax * stripe, stripe)],
                    parts_hbm.at[c_ax, pl.ds(s_ax * stripe, stripe)])
    if tail:
        @pl.when(s_ax == 0)
        def _():
            pltpu.sync_copy(acc.at[pl.ds(NS * stripe, tail)],
                            parts_hbm.at[c_ax, pl.ds(NS * stripe, tail)])


def _edge_scatter(h2, dst, zeros, N):
    e, d = h2.shape
    mesh = plsc.VectorSubcoreMesh(core_axis_name="c", subcore_axis_name="s")
    return pl.kernel(
        _scatter_body,
        out_type=jax.ShapeDtypeStruct((NC, N, d), jnp.float32),
        mesh=mesh,
        scratch_types=[
            pltpu.VMEM((CHUNK,), jnp.int32),
            pltpu.VMEM((CHUNK,), jnp.int32),
            pltpu.VMEM((CHUNK,), jnp.int32),
            pltpu.VMEM((CHUNK,), jnp.int32),
            pltpu.VMEM((4, CHUNK, d), jnp.float32),
            pltpu.VMEM_SHARED((N, d), jnp.float32),
            pltpu.SemaphoreType.DMA,
            pltpu.SemaphoreType.DMA,
            pltpu.SemaphoreType.DMA,
            pltpu.SemaphoreType.DMA,
            pltpu.SemaphoreType.DMA,
            pltpu.SemaphoreType.DMA,
            pltpu.SemaphoreType.DMA,
            pltpu.SemaphoreType.DMA,
        ],
    )(h2, dst, zeros)


def _combine_body(p0_ref, p1_ref, o_ref):
    o_ref[...] = p0_ref[...] + p1_ref[...]


def _combine(parts, block_n=1000):
    _, n, d = parts.shape
    grid = n // block_n
    return pl.pallas_call(
        _combine_body,
        grid=(grid,),
        in_specs=[
            pl.BlockSpec((block_n, d), lambda i: (i, 0)),
            pl.BlockSpec((block_n, d), lambda i: (i, 0)),
        ],
        out_specs=pl.BlockSpec((block_n, d), lambda i: (i, 0)),
        out_shape=jax.ShapeDtypeStruct((n, d), jnp.float32),
    )(parts[0], parts[1])


def kernel(x, edge_index, W1, b1, W2, b2):
    n, d = x.shape
    e = edge_index.shape[1]
    src = edge_index[0]
    dst = edge_index[1]
    A, B = _node_mm(x, W1, b1.reshape(1, d))
    h1 = _edge_gather(A, B, dst, src, e)
    h2 = _edge_mm(h1, W2, b2.reshape(1, d))
    zeros = jnp.zeros((n, d), jnp.float32)
    parts = _edge_scatter(h2, dst, zeros, n)
    return _combine(parts)


# speedup vs baseline: 5.8749x; 1.0015x over previous
import functools

import jax
import jax.numpy as jnp
from jax import lax
from jax.experimental import pallas as pl
from jax.experimental.pallas import tpu as pltpu
from jax.experimental.pallas import tpu_sc as plsc

NC, NS, LANES = 2, 16, 16
NW = NC * NS
CHUNK = 80


def _node_mm_body(x_ref, w1_ref, b1_ref, a_ref, b_ref):
    xb = x_ref[...]
    d = x_ref.shape[1]
    w1a = w1_ref[:d, :]
    w1b = w1_ref[d:, :]
    a_ref[...] = jnp.dot(xb, w1a - w1b, preferred_element_type=jnp.float32) + b1_ref[...]
    b_ref[...] = jnp.dot(xb, w1b, preferred_element_type=jnp.float32)


def _node_mm(x, W1, b1, block_n=1000):
    n, d = x.shape
    grid = n // block_n
    return pl.pallas_call(
        _node_mm_body,
        grid=(grid,),
        in_specs=[
            pl.BlockSpec((block_n, d), lambda i: (i, 0)),
            pl.BlockSpec((2 * d, d), lambda i: (0, 0)),
            pl.BlockSpec((1, d), lambda i: (0, 0)),
        ],
        out_specs=[
            pl.BlockSpec((block_n, d), lambda i: (i, 0)),
            pl.BlockSpec((block_n, d), lambda i: (i, 0)),
        ],
        out_shape=[
            jax.ShapeDtypeStruct((n, d), jnp.float32),
            jax.ShapeDtypeStruct((n, d), jnp.float32),
        ],
    )(x, W1, b1)


def _gather_body(CHUNK, a_hbm, b_hbm, dst_hbm, src_hbm, h1_hbm,
                 dsti, srci, arows, brows, hrows,
                 gsem0, gsem1, wsem0, wsem1):
    e, d = h1_hbm.shape
    per_w = e // NW
    n_chunks = per_w // CHUNK
    assert n_chunks % 2 == 1
    pairs = (n_chunks - 1) // 2
    c_ax = lax.axis_index("c")
    s_ax = lax.axis_index("s")
    wid = s_ax * NC + c_ax
    base_w = wid * per_w
    gsems = (gsem0, gsem1)
    wsems = (wsem0, wsem1)

    pltpu.sync_copy(dst_hbm.at[pl.ds(base_w, per_w)], dsti)
    pltpu.sync_copy(src_hbm.at[pl.ds(base_w, per_w)], srci)

    def issue_gather(ci, b):
        i0 = ci * CHUNK
        pltpu.async_copy(a_hbm.at[dsti.at[pl.ds(i0, CHUNK)]], arows.at[b], gsems[b])
        pltpu.async_copy(b_hbm.at[srci.at[pl.ds(i0, CHUNK)]], brows.at[b], gsems[b])

    def wait_gather(b):
        pltpu.make_async_copy(a_hbm.at[pl.ds(0, CHUNK)], arows.at[b], gsems[b]).wait()
        pltpu.make_async_copy(b_hbm.at[pl.ds(0, CHUNK)], brows.at[b], gsems[b]).wait()

    def wait_wb(b):
        pltpu.make_async_copy(hrows.at[b], h1_hbm.at[pl.ds(0, CHUNK)], wsems[b]).wait()

    def compute(b):
        ar = arows.at[b]
        br = brows.at[b]
        hr = hrows.at[b]

        def row_body(r, rc):
            r8 = r * 8
            for rr in range(8):
                for g in range(d // LANES):
                    sl = pl.ds(g * LANES, LANES)
                    v = ar[r8 + rr, sl] + br[r8 + rr, sl]
                    hr[r8 + rr, sl] = jnp.maximum(v, 0.01 * v)
            return rc

        lax.fori_loop(0, CHUNK // 8, row_body, 0)

    issue_gather(0, 0)
    issue_gather(1, 1)

    def pair_body(p, carry):
        for b in range(2):
            c = p * 2 + b
            wait_gather(b)

            @pl.when(p >= 1)
            def _():
                wait_wb(b)

            compute(b)
            pltpu.async_copy(hrows.at[b], h1_hbm.at[pl.ds(base_w + c * CHUNK, CHUNK)],
                             wsems[b])
            if b == 0:
                issue_gather(c + 2, b)
            else:
                @pl.when(p < pairs - 1)
                def _():
                    issue_gather(c + 2, b)
        return carry

    lax.fori_loop(0, pairs, pair_body, 0)

    c = n_chunks - 1
    wait_gather(0)
    wait_wb(0)
    compute(0)
    pltpu.sync_copy(hrows.at[0], h1_hbm.at[pl.ds(base_w + c * CHUNK, CHUNK)])
    wait_wb(1)


def _edge_gather(A, B, dst, src, E, chunk=CHUNK):
    d = A.shape[1]
    per_w = E // NW
    mesh = plsc.VectorSubcoreMesh(core_axis_name="c", subcore_axis_name="s")
    return pl.kernel(
        functools.partial(_gather_body, chunk),
        out_type=jax.ShapeDtypeStruct((E, d), jnp.float32),
        mesh=mesh,
        scratch_types=[
            pltpu.VMEM((per_w,), jnp.int32),
            pltpu.VMEM((per_w,), jnp.int32),
            pltpu.VMEM((2, chunk, d), jnp.float32),
            pltpu.VMEM((2, chunk, d), jnp.float32),
            pltpu.VMEM((2, chunk, d), jnp.float32),
            pltpu.SemaphoreType.DMA,
            pltpu.SemaphoreType.DMA,
            pltpu.SemaphoreType.DMA,
            pltpu.SemaphoreType.DMA,
        ],
    )(A, B, dst, src)


def _edge_mm_body(h1_ref, w2_ref, b2_ref, h2_ref):
    h = jnp.dot(h1_ref[...], w2_ref[...], preferred_element_type=jnp.float32) + b2_ref[...]
    h2_ref[...] = jnp.maximum(h, 0.01 * h)


def _edge_mm(h1, W2, b2, block_e=4000):
    e, d = h1.shape
    grid = e // block_e
    return pl.pallas_call(
        _edge_mm_body,
        grid=(grid,),
        in_specs=[
            pl.BlockSpec((block_e, d), lambda i: (i, 0)),
            pl.BlockSpec((d, d), lambda i: (0, 0)),
            pl.BlockSpec((1, d), lambda i: (0, 0)),
        ],
        out_specs=pl.BlockSpec((block_e, d), lambda i: (i, 0)),
        out_shape=jax.ShapeDtypeStruct((e, d), jnp.float32),
    )(h1, W2, b2)


def _scatter_body(CHUNK, h2_hbm, dst_hbm, zeros_hbm, parts_hbm,
                  dstv0, dstv1, dstv2, dstv3, rows, acc,
                  lsem0, lsem1, lsem2, lsem3,
                  ssem0, ssem1, ssem2, ssem3):
    e, d = h2_hbm.shape
    n = zeros_hbm.shape[0]
    per_w = e // NW
    n_chunks = per_w // CHUNK
    assert n_chunks % 4 == 1
    quads = n_chunks // 4
    stripe = (n // NS) // 8 * 8
    tail = n - NS * stripe
    c_ax = lax.axis_index("c")
    s_ax = lax.axis_index("s")
    wid = s_ax * NC + c_ax
    base_w = wid * per_w
    dstvs = (dstv0, dstv1, dstv2, dstv3)
    lsems = (lsem0, lsem1, lsem2, lsem3)
    ssems = (ssem0, ssem1, ssem2, ssem3)

    pltpu.sync_copy(zeros_hbm.at[pl.ds(s_ax * stripe, stripe)],
                    acc.at[pl.ds(s_ax * stripe, stripe)])
    if tail:
        @pl.when(s_ax == 0)
        def _():
            pltpu.sync_copy(zeros_hbm.at[pl.ds(NS * stripe, tail)],
                            acc.at[pl.ds(NS * stripe, tail)])
    plsc.subcore_barrier()

    def issue_load(ci, b):
        base = base_w + ci * CHUNK
        pltpu.async_copy(dst_hbm.at[pl.ds(base, CHUNK)], dstvs[b], lsems[b])
        pltpu.async_copy(h2_hbm.at[pl.ds(base, CHUNK)], rows.at[b], lsems[b])

    def wait_load(b):
        pltpu.make_async_copy(dst_hbm.at[pl.ds(0, CHUNK)], dstvs[b], lsems[b]).wait()
        pltpu.make_async_copy(h2_hbm.at[pl.ds(0, CHUNK)], rows.at[b], lsems[b]).wait()

    def scatter_go(b):
        pltpu.async_copy(rows.at[b], acc.at[dstvs[b]], ssems[b], add=True)

    def wait_scatter(b):
        pltpu.make_async_copy(h2_hbm.at[pl.ds(0, CHUNK)], rows.at[b], ssems[b]).wait()

    issue_load(0, 0)
    issue_load(1, 1)

    def quad_body(q, carry):
        for i in range(4):
            v = q * 4 + i
            wait_load(i)
            scatter_go(i)
            bw = (i + 2) % 4
            if i < 2:
                @pl.when(q >= 1)
                def _():
                    wait_scatter(bw)

                issue_load(v + 2, bw)
            else:
                @pl.when(v + 2 < n_chunks)
                def _():
                    wait_scatter(bw)
                    issue_load(v + 2, bw)
        return carry

    lax.fori_loop(0, quads, quad_body, 0)
    wait_load(0)
    scatter_go(0)
    for b in range(4):
        wait_scatter(b)

    plsc.subcore_barrier()
    pltpu.sync_copy(acc.at[pl.ds(s_ax * stripe, stripe)],
                    parts_hbm.at[c_ax, pl.ds(s_ax * stripe, stripe)])
    if tail:
        @pl.when(s_ax == 0)
        def _():
            pltpu.sync_copy(acc.at[pl.ds(NS * stripe, tail)],
                            parts_hbm.at[c_ax, pl.ds(NS * stripe, tail)])


def _edge_scatter(h2, dst, zeros, N, chunk=CHUNK):
    e, d = h2.shape
    mesh = plsc.VectorSubcoreMesh(core_axis_name="c", subcore_axis_name="s")
    return pl.kernel(
        functools.partial(_scatter_body, chunk),
        out_type=jax.ShapeDtypeStruct((NC, N, d), jnp.float32),
        mesh=mesh,
        scratch_types=[
            pltpu.VMEM((chunk,), jnp.int32),
            pltpu.VMEM((chunk,), jnp.int32),
            pltpu.VMEM((chunk,), jnp.int32),
            pltpu.VMEM((chunk,), jnp.int32),
            pltpu.VMEM((4, chunk, d), jnp.float32),
            pltpu.VMEM_SHARED((N, d), jnp.float32),
            pltpu.SemaphoreType.DMA,
            pltpu.SemaphoreType.DMA,
            pltpu.SemaphoreType.DMA,
            pltpu.SemaphoreType.DMA,
            pltpu.SemaphoreType.DMA,
            pltpu.SemaphoreType.DMA,
            pltpu.SemaphoreType.DMA,
            pltpu.SemaphoreType.DMA,
        ],
    )(h2, dst, zeros)


def _combine_body(*refs):
    o_ref = refs[-1]
    acc = refs[0][...]
    for r in refs[1:-1]:
        acc = acc + r[...]
    o_ref[...] = acc


def _combine(parts_list, block_n=1000):
    terms = [p[i] for p in parts_list for i in range(p.shape[0])]
    n, d = terms[0].shape
    grid = n // block_n
    return pl.pallas_call(
        _combine_body,
        grid=(grid,),
        in_specs=[pl.BlockSpec((block_n, d), lambda i: (i, 0)) for _ in terms],
        out_specs=pl.BlockSpec((block_n, d), lambda i: (i, 0)),
        out_shape=jax.ShapeDtypeStruct((n, d), jnp.float32),
    )(*terms)


def kernel(x, edge_index, W1, b1, W2, b2):
    n, d = x.shape
    e = edge_index.shape[1]
    src = edge_index[0]
    dst = edge_index[1]
    A, B = _node_mm(x, W1, b1.reshape(1, d))
    zeros = jnp.zeros((n, d), jnp.float32)
    h1 = _edge_gather(A, B, dst, src, e)
    h2 = _edge_mm(h1, W2, b2.reshape(1, d))
    parts = _edge_scatter(h2, dst, zeros, n)
    return _combine([parts])


# speedup vs baseline: 6.0630x; 1.0320x over previous
import functools

import jax
import jax.numpy as jnp
from jax import lax
from jax.experimental import pallas as pl
from jax.experimental.pallas import tpu as pltpu
from jax.experimental.pallas import tpu_sc as plsc

NC, NS, LANES = 2, 16, 16
NW = NC * NS
CHUNK = 80


def _node_mm_body(x_ref, w1_ref, b1_ref, a_ref, b_ref):
    xb = x_ref[...]
    d = x_ref.shape[1]
    w1a = w1_ref[:d, :]
    w1b = w1_ref[d:, :]
    a_ref[...] = jnp.dot(xb, w1a - w1b, preferred_element_type=jnp.float32) + b1_ref[...]
    b_ref[...] = jnp.dot(xb, w1b, preferred_element_type=jnp.float32)


def _node_mm(x, W1, b1, block_n=1000):
    n, d = x.shape
    grid = n // block_n
    return pl.pallas_call(
        _node_mm_body,
        grid=(grid,),
        in_specs=[
            pl.BlockSpec((block_n, d), lambda i: (i, 0)),
            pl.BlockSpec((2 * d, d), lambda i: (0, 0)),
            pl.BlockSpec((1, d), lambda i: (0, 0)),
        ],
        out_specs=[
            pl.BlockSpec((block_n, d), lambda i: (i, 0)),
            pl.BlockSpec((block_n, d), lambda i: (i, 0)),
        ],
        out_shape=[
            jax.ShapeDtypeStruct((n, d), jnp.float32),
            jax.ShapeDtypeStruct((n, d), jnp.float32),
        ],
    )(x, W1, b1)


def _gather_body(CHUNK, a_hbm, b_hbm, dst_hbm, src_hbm, h1_hbm,
                 dsti, srci, arows, brows, hrows,
                 gsem0, gsem1, wsem0, wsem1):
    e, d = h1_hbm.shape
    per_w = e // NW
    n_chunks = per_w // CHUNK
    assert n_chunks % 2 == 1
    pairs = (n_chunks - 1) // 2
    c_ax = lax.axis_index("c")
    s_ax = lax.axis_index("s")
    wid = s_ax * NC + c_ax
    base_w = wid * per_w
    gsems = (gsem0, gsem1)
    wsems = (wsem0, wsem1)

    pltpu.sync_copy(dst_hbm.at[pl.ds(base_w, per_w)], dsti)
    pltpu.sync_copy(src_hbm.at[pl.ds(base_w, per_w)], srci)

    def issue_gather(ci, b):
        i0 = ci * CHUNK
        pltpu.async_copy(a_hbm.at[dsti.at[pl.ds(i0, CHUNK)]], arows.at[b], gsems[b])
        pltpu.async_copy(b_hbm.at[srci.at[pl.ds(i0, CHUNK)]], brows.at[b], gsems[b])

    def wait_gather(b):
        pltpu.make_async_copy(a_hbm.at[pl.ds(0, CHUNK)], arows.at[b], gsems[b]).wait()
        pltpu.make_async_copy(b_hbm.at[pl.ds(0, CHUNK)], brows.at[b], gsems[b]).wait()

    def wait_wb(b):
        pltpu.make_async_copy(hrows.at[b], h1_hbm.at[pl.ds(0, CHUNK)], wsems[b]).wait()

    def compute(b):
        ar = arows.at[b]
        br = brows.at[b]
        hr = hrows.at[b]

        def row_body(r, rc):
            r8 = r * 8
            for rr in range(8):
                for g in range(d // LANES):
                    sl = pl.ds(g * LANES, LANES)
                    v = ar[r8 + rr, sl] + br[r8 + rr, sl]
                    hr[r8 + rr, sl] = jnp.maximum(v, 0.01 * v)
            return rc

        lax.fori_loop(0, CHUNK // 8, row_body, 0)

    issue_gather(0, 0)
    issue_gather(1, 1)

    def pair_body(p, carry):
        for b in range(2):
            c = p * 2 + b
            wait_gather(b)

            @pl.when(p >= 1)
            def _():
                wait_wb(b)

            compute(b)
            pltpu.async_copy(hrows.at[b], h1_hbm.at[pl.ds(base_w + c * CHUNK, CHUNK)],
                             wsems[b])
            if b == 0:
                issue_gather(c + 2, b)
            else:
                @pl.when(p < pairs - 1)
                def _():
                    issue_gather(c + 2, b)
        return carry

    lax.fori_loop(0, pairs, pair_body, 0)

    c = n_chunks - 1
    wait_gather(0)
    wait_wb(0)
    compute(0)
    pltpu.sync_copy(hrows.at[0], h1_hbm.at[pl.ds(base_w + c * CHUNK, CHUNK)])
    wait_wb(1)


def _edge_gather(A, B, dst, src, E, chunk=CHUNK):
    d = A.shape[1]
    per_w = E // NW
    mesh = plsc.VectorSubcoreMesh(core_axis_name="c", subcore_axis_name="s")
    return pl.kernel(
        functools.partial(_gather_body, chunk),
        out_type=jax.ShapeDtypeStruct((E, d), jnp.float32),
        mesh=mesh,
        scratch_types=[
            pltpu.VMEM((per_w,), jnp.int32),
            pltpu.VMEM((per_w,), jnp.int32),
            pltpu.VMEM((2, chunk, d), jnp.float32),
            pltpu.VMEM((2, chunk, d), jnp.float32),
            pltpu.VMEM((2, chunk, d), jnp.float32),
            pltpu.SemaphoreType.DMA,
            pltpu.SemaphoreType.DMA,
            pltpu.SemaphoreType.DMA,
            pltpu.SemaphoreType.DMA,
        ],
    )(A, B, dst, src)


def _edge_mm_body(h1_ref, w2_ref, b2_ref, h2_ref):
    h = jnp.dot(h1_ref[...], w2_ref[...], preferred_element_type=jnp.float32) + b2_ref[...]
    h2_ref[...] = jnp.maximum(h, 0.01 * h)


def _edge_mm(h1, W2, b2, block_e=8000):
    e, d = h1.shape
    grid = e // block_e
    return pl.pallas_call(
        _edge_mm_body,
        grid=(grid,),
        in_specs=[
            pl.BlockSpec((block_e, d), lambda i: (i, 0)),
            pl.BlockSpec((d, d), lambda i: (0, 0)),
            pl.BlockSpec((1, d), lambda i: (0, 0)),
        ],
        out_specs=pl.BlockSpec((block_e, d), lambda i: (i, 0)),
        out_shape=jax.ShapeDtypeStruct((e, d), jnp.float32),
    )(h1, W2, b2)


def _scatter_body(CHUNK, h2_hbm, dst_hbm, zeros_hbm, parts_hbm,
                  dstv0, dstv1, dstv2, dstv3, rows, acc,
                  lsem0, lsem1, lsem2, lsem3,
                  ssem0, ssem1, ssem2, ssem3):
    e, d = h2_hbm.shape
    n = zeros_hbm.shape[0]
    per_w = e // NW
    n_chunks = per_w // CHUNK
    assert n_chunks % 4 == 1
    quads = n_chunks // 4
    stripe = (n // NS) // 8 * 8
    tail = n - NS * stripe
    c_ax = lax.axis_index("c")
    s_ax = lax.axis_index("s")
    wid = s_ax * NC + c_ax
    base_w = wid * per_w
    dstvs = (dstv0, dstv1, dstv2, dstv3)
    lsems = (lsem0, lsem1, lsem2, lsem3)
    ssems = (ssem0, ssem1, ssem2, ssem3)

    pltpu.sync_copy(zeros_hbm.at[pl.ds(s_ax * stripe, stripe)],
                    acc.at[pl.ds(s_ax * stripe, stripe)])
    if tail:
        @pl.when(s_ax == 0)
        def _():
            pltpu.sync_copy(zeros_hbm.at[pl.ds(NS * stripe, tail)],
                            acc.at[pl.ds(NS * stripe, tail)])
    plsc.subcore_barrier()

    def issue_load(ci, b):
        base = base_w + ci * CHUNK
        pltpu.async_copy(dst_hbm.at[pl.ds(base, CHUNK)], dstvs[b], lsems[b])
        pltpu.async_copy(h2_hbm.at[pl.ds(base, CHUNK)], rows.at[b], lsems[b])

    def wait_load(b):
        pltpu.make_async_copy(dst_hbm.at[pl.ds(0, CHUNK)], dstvs[b], lsems[b]).wait()
        pltpu.make_async_copy(h2_hbm.at[pl.ds(0, CHUNK)], rows.at[b], lsems[b]).wait()

    def scatter_go(b):
        pltpu.async_copy(rows.at[b], acc.at[dstvs[b]], ssems[b], add=True)

    def wait_scatter(b):
        pltpu.make_async_copy(h2_hbm.at[pl.ds(0, CHUNK)], rows.at[b], ssems[b]).wait()

    issue_load(0, 0)
    issue_load(1, 1)

    def quad_body(q, carry):
        for i in range(4):
            v = q * 4 + i
            wait_load(i)
            scatter_go(i)
            bw = (i + 2) % 4
            if i < 2:
                @pl.when(q >= 1)
                def _():
                    wait_scatter(bw)

                issue_load(v + 2, bw)
            else:
                @pl.when(v + 2 < n_chunks)
                def _():
                    wait_scatter(bw)
                    issue_load(v + 2, bw)
        return carry

    lax.fori_loop(0, quads, quad_body, 0)
    wait_load(0)
    scatter_go(0)
    for b in range(4):
        wait_scatter(b)

    plsc.subcore_barrier()
    pltpu.sync_copy(acc.at[pl.ds(s_ax * stripe, stripe)],
                    parts_hbm.at[c_ax, pl.ds(s_ax * stripe, stripe)])
    if tail:
        @pl.when(s_ax == 0)
        def _():
            pltpu.sync_copy(acc.at[pl.ds(NS * stripe, tail)],
                            parts_hbm.at[c_ax, pl.ds(NS * stripe, tail)])


def _edge_scatter(h2, dst, zeros, N, chunk=CHUNK):
    e, d = h2.shape
    mesh = plsc.VectorSubcoreMesh(core_axis_name="c", subcore_axis_name="s")
    return pl.kernel(
        functools.partial(_scatter_body, chunk),
        out_type=jax.ShapeDtypeStruct((NC, N, d), jnp.float32),
        mesh=mesh,
        scratch_types=[
            pltpu.VMEM((chunk,), jnp.int32),
            pltpu.VMEM((chunk,), jnp.int32),
            pltpu.VMEM((chunk,), jnp.int32),
            pltpu.VMEM((chunk,), jnp.int32),
            pltpu.VMEM((4, chunk, d), jnp.float32),
            pltpu.VMEM_SHARED((N, d), jnp.float32),
            pltpu.SemaphoreType.DMA,
            pltpu.SemaphoreType.DMA,
            pltpu.SemaphoreType.DMA,
            pltpu.SemaphoreType.DMA,
            pltpu.SemaphoreType.DMA,
            pltpu.SemaphoreType.DMA,
            pltpu.SemaphoreType.DMA,
            pltpu.SemaphoreType.DMA,
        ],
    )(h2, dst, zeros)


def _combine_body(*refs):
    o_ref = refs[-1]
    acc = refs[0][...]
    for r in refs[1:-1]:
        acc = acc + r[...]
    o_ref[...] = acc


def _combine(parts_list, block_n=1000):
    terms = [p[i] for p in parts_list for i in range(p.shape[0])]
    n, d = terms[0].shape
    grid = n // block_n
    return pl.pallas_call(
        _combine_body,
        grid=(grid,),
        in_specs=[pl.BlockSpec((block_n, d), lambda i: (i, 0)) for _ in terms],
        out_specs=pl.BlockSpec((block_n, d), lambda i: (i, 0)),
        out_shape=jax.ShapeDtypeStruct((n, d), jnp.float32),
    )(*terms)


def kernel(x, edge_index, W1, b1, W2, b2):
    n, d = x.shape
    e = edge_index.shape[1]
    src = edge_index[0]
    dst = edge_index[1]
    A, B = _node_mm(x, W1, b1.reshape(1, d))
    zeros = jnp.zeros((n, d), jnp.float32)
    h1 = _edge_gather(A, B, dst, src, e)
    h2 = _edge_mm(h1, W2, b2.reshape(1, d))
    parts = _edge_scatter(h2, dst, zeros, n)
    return _combine([parts])


# speedup vs baseline: 6.1735x; 1.0182x over previous
import functools

import jax
import jax.numpy as jnp
from jax import lax
from jax.experimental import pallas as pl
from jax.experimental.pallas import tpu as pltpu
from jax.experimental.pallas import tpu_sc as plsc

NC, NS, LANES = 2, 16, 16
NW = NC * NS
CHUNK = 80


def _node_mm_body(x_ref, w1_ref, b1_ref, a_ref, b_ref):
    xb = x_ref[...]
    d = x_ref.shape[1]
    w1a = w1_ref[:d, :]
    w1b = w1_ref[d:, :]
    a_ref[...] = jnp.dot(xb, w1a - w1b, preferred_element_type=jnp.float32) + b1_ref[...]
    b_ref[...] = jnp.dot(xb, w1b, preferred_element_type=jnp.float32)


def _node_mm(x, W1, b1, block_n=2000):
    n, d = x.shape
    grid = n // block_n
    return pl.pallas_call(
        _node_mm_body,
        grid=(grid,),
        in_specs=[
            pl.BlockSpec((block_n, d), lambda i: (i, 0)),
            pl.BlockSpec((2 * d, d), lambda i: (0, 0)),
            pl.BlockSpec((1, d), lambda i: (0, 0)),
        ],
        out_specs=[
            pl.BlockSpec((block_n, d), lambda i: (i, 0)),
            pl.BlockSpec((block_n, d), lambda i: (i, 0)),
        ],
        out_shape=[
            jax.ShapeDtypeStruct((n, d), jnp.float32),
            jax.ShapeDtypeStruct((n, d), jnp.float32),
        ],
    )(x, W1, b1)


def _gather_body(CHUNK, a_hbm, b_hbm, dst_hbm, src_hbm, h1_hbm,
                 dsti, srci, arows, brows, hrows,
                 gsem0, gsem1, wsem0, wsem1):
    e, d = h1_hbm.shape
    per_w = e // NW
    n_chunks = per_w // CHUNK
    assert n_chunks % 2 == 1
    pairs = (n_chunks - 1) // 2
    c_ax = lax.axis_index("c")
    s_ax = lax.axis_index("s")
    wid = s_ax * NC + c_ax
    base_w = wid * per_w
    gsems = (gsem0, gsem1)
    wsems = (wsem0, wsem1)

    pltpu.sync_copy(dst_hbm.at[pl.ds(base_w, per_w)], dsti)
    pltpu.sync_copy(src_hbm.at[pl.ds(base_w, per_w)], srci)

    def issue_gather(ci, b):
        i0 = ci * CHUNK
        pltpu.async_copy(a_hbm.at[dsti.at[pl.ds(i0, CHUNK)]], arows.at[b], gsems[b])
        pltpu.async_copy(b_hbm.at[srci.at[pl.ds(i0, CHUNK)]], brows.at[b], gsems[b])

    def wait_gather(b):
        pltpu.make_async_copy(a_hbm.at[pl.ds(0, CHUNK)], arows.at[b], gsems[b]).wait()
        pltpu.make_async_copy(b_hbm.at[pl.ds(0, CHUNK)], brows.at[b], gsems[b]).wait()

    def wait_wb(b):
        pltpu.make_async_copy(hrows.at[b], h1_hbm.at[pl.ds(0, CHUNK)], wsems[b]).wait()

    def compute(b):
        ar = arows.at[b]
        br = brows.at[b]
        hr = hrows.at[b]

        def row_body(r, rc):
            r8 = r * 8
            for rr in range(8):
                for g in range(d // LANES):
                    sl = pl.ds(g * LANES, LANES)
                    v = ar[r8 + rr, sl] + br[r8 + rr, sl]
                    hr[r8 + rr, sl] = jnp.maximum(v, 0.01 * v)
            return rc

        lax.fori_loop(0, CHUNK // 8, row_body, 0)

    issue_gather(0, 0)
    issue_gather(1, 1)

    def pair_body(p, carry):
        for b in range(2):
            c = p * 2 + b
            wait_gather(b)

            @pl.when(p >= 1)
            def _():
                wait_wb(b)

            compute(b)
            pltpu.async_copy(hrows.at[b], h1_hbm.at[pl.ds(base_w + c * CHUNK, CHUNK)],
                             wsems[b])
            if b == 0:
                issue_gather(c + 2, b)
            else:
                @pl.when(p < pairs - 1)
                def _():
                    issue_gather(c + 2, b)
        return carry

    lax.fori_loop(0, pairs, pair_body, 0)

    c = n_chunks - 1
    wait_gather(0)
    wait_wb(0)
    compute(0)
    pltpu.sync_copy(hrows.at[0], h1_hbm.at[pl.ds(base_w + c * CHUNK, CHUNK)])
    wait_wb(1)


def _edge_gather(A, B, dst, src, E, chunk=CHUNK):
    d = A.shape[1]
    per_w = E // NW
    mesh = plsc.VectorSubcoreMesh(core_axis_name="c", subcore_axis_name="s")
    return pl.kernel(
        functools.partial(_gather_body, chunk),
        out_type=jax.ShapeDtypeStruct((E, d), jnp.float32),
        mesh=mesh,
        scratch_types=[
            pltpu.VMEM((per_w,), jnp.int32),
            pltpu.VMEM((per_w,), jnp.int32),
            pltpu.VMEM((2, chunk, d), jnp.float32),
            pltpu.VMEM((2, chunk, d), jnp.float32),
            pltpu.VMEM((2, chunk, d), jnp.float32),
            pltpu.SemaphoreType.DMA,
            pltpu.SemaphoreType.DMA,
            pltpu.SemaphoreType.DMA,
            pltpu.SemaphoreType.DMA,
        ],
    )(A, B, dst, src)


def _edge_mm_body(h1_ref, w2_ref, b2_ref, h2_ref):
    h = jnp.dot(h1_ref[...], w2_ref[...], preferred_element_type=jnp.float32) + b2_ref[...]
    h2_ref[...] = jnp.maximum(h, 0.01 * h)


def _edge_mm(h1, W2, b2, block_e=16000):
    e, d = h1.shape
    grid = e // block_e
    return pl.pallas_call(
        _edge_mm_body,
        grid=(grid,),
        in_specs=[
            pl.BlockSpec((block_e, d), lambda i: (i, 0)),
            pl.BlockSpec((d, d), lambda i: (0, 0)),
            pl.BlockSpec((1, d), lambda i: (0, 0)),
        ],
        out_specs=pl.BlockSpec((block_e, d), lambda i: (i, 0)),
        out_shape=jax.ShapeDtypeStruct((e, d), jnp.float32),
    )(h1, W2, b2)


def _scatter_body(CHUNK, h2_hbm, dst_hbm, zeros_hbm, parts_hbm,
                  dstv0, dstv1, dstv2, dstv3, rows, acc,
                  lsem0, lsem1, lsem2, lsem3,
                  ssem0, ssem1, ssem2, ssem3):
    e, d = h2_hbm.shape
    n = zeros_hbm.shape[0]
    per_w = e // NW
    n_chunks = per_w // CHUNK
    assert n_chunks % 4 == 1
    quads = n_chunks // 4
    stripe = (n // NS) // 8 * 8
    tail = n - NS * stripe
    c_ax = lax.axis_index("c")
    s_ax = lax.axis_index("s")
    wid = s_ax * NC + c_ax
    base_w = wid * per_w
    dstvs = (dstv0, dstv1, dstv2, dstv3)
    lsems = (lsem0, lsem1, lsem2, lsem3)
    ssems = (ssem0, ssem1, ssem2, ssem3)

    pltpu.sync_copy(zeros_hbm.at[pl.ds(s_ax * stripe, stripe)],
                    acc.at[pl.ds(s_ax * stripe, stripe)])
    if tail:
        @pl.when(s_ax == 0)
        def _():
            pltpu.sync_copy(zeros_hbm.at[pl.ds(NS * stripe, tail)],
                            acc.at[pl.ds(NS * stripe, tail)])
    plsc.subcore_barrier()

    def issue_load(ci, b):
        base = base_w + ci * CHUNK
        pltpu.async_copy(dst_hbm.at[pl.ds(base, CHUNK)], dstvs[b], lsems[b])
        pltpu.async_copy(h2_hbm.at[pl.ds(base, CHUNK)], rows.at[b], lsems[b])

    def wait_load(b):
        pltpu.make_async_copy(dst_hbm.at[pl.ds(0, CHUNK)], dstvs[b], lsems[b]).wait()
        pltpu.make_async_copy(h2_hbm.at[pl.ds(0, CHUNK)], rows.at[b], lsems[b]).wait()

    def scatter_go(b):
        pltpu.async_copy(rows.at[b], acc.at[dstvs[b]], ssems[b], add=True)

    def wait_scatter(b):
        pltpu.make_async_copy(h2_hbm.at[pl.ds(0, CHUNK)], rows.at[b], ssems[b]).wait()

    issue_load(0, 0)
    issue_load(1, 1)

    def quad_body(q, carry):
        for i in range(4):
            v = q * 4 + i
            wait_load(i)
            scatter_go(i)
            bw = (i + 2) % 4
            if i < 2:
                @pl.when(q >= 1)
                def _():
                    wait_scatter(bw)

                issue_load(v + 2, bw)
            else:
                @pl.when(v + 2 < n_chunks)
                def _():
                    wait_scatter(bw)
                    issue_load(v + 2, bw)
        return carry

    lax.fori_loop(0, quads, quad_body, 0)
    wait_load(0)
    scatter_go(0)
    for b in range(4):
        wait_scatter(b)

    plsc.subcore_barrier()
    pltpu.sync_copy(acc.at[pl.ds(s_ax * stripe, stripe)],
                    parts_hbm.at[c_ax, pl.ds(s_ax * stripe, stripe)])
    if tail:
        @pl.when(s_ax == 0)
        def _():
            pltpu.sync_copy(acc.at[pl.ds(NS * stripe, tail)],
                            parts_hbm.at[c_ax, pl.ds(NS * stripe, tail)])


def _edge_scatter(h2, dst, zeros, N, chunk=CHUNK):
    e, d = h2.shape
    mesh = plsc.VectorSubcoreMesh(core_axis_name="c", subcore_axis_name="s")
    return pl.kernel(
        functools.partial(_scatter_body, chunk),
        out_type=jax.ShapeDtypeStruct((NC, N, d), jnp.float32),
        mesh=mesh,
        scratch_types=[
            pltpu.VMEM((chunk,), jnp.int32),
            pltpu.VMEM((chunk,), jnp.int32),
            pltpu.VMEM((chunk,), jnp.int32),
            pltpu.VMEM((chunk,), jnp.int32),
            pltpu.VMEM((4, chunk, d), jnp.float32),
            pltpu.VMEM_SHARED((N, d), jnp.float32),
            pltpu.SemaphoreType.DMA,
            pltpu.SemaphoreType.DMA,
            pltpu.SemaphoreType.DMA,
            pltpu.SemaphoreType.DMA,
            pltpu.SemaphoreType.DMA,
            pltpu.SemaphoreType.DMA,
            pltpu.SemaphoreType.DMA,
            pltpu.SemaphoreType.DMA,
        ],
    )(h2, dst, zeros)


def _combine_body(*refs):
    o_ref = refs[-1]
    acc = refs[0][...]
    for r in refs[1:-1]:
        acc = acc + r[...]
    o_ref[...] = acc


def _combine(parts_list, block_n=2000):
    terms = [p[i] for p in parts_list for i in range(p.shape[0])]
    n, d = terms[0].shape
    grid = n // block_n
    return pl.pallas_call(
        _combine_body,
        grid=(grid,),
        in_specs=[pl.BlockSpec((block_n, d), lambda i: (i, 0)) for _ in terms],
        out_specs=pl.BlockSpec((block_n, d), lambda i: (i, 0)),
        out_shape=jax.ShapeDtypeStruct((n, d), jnp.float32),
    )(*terms)


def kernel(x, edge_index, W1, b1, W2, b2):
    n, d = x.shape
    e = edge_index.shape[1]
    src = edge_index[0]
    dst = edge_index[1]
    A, B = _node_mm(x, W1, b1.reshape(1, d))
    zeros = jnp.zeros((n, d), jnp.float32)
    h1 = _edge_gather(A, B, dst, src, e)
    h2 = _edge_mm(h1, W2, b2.reshape(1, d))
    parts = _edge_scatter(h2, dst, zeros, n)
    return _combine([parts])


# speedup vs baseline: 6.2878x; 1.0185x over previous
import functools

import jax
import jax.numpy as jnp
from jax import lax
from jax.experimental import pallas as pl
from jax.experimental.pallas import tpu as pltpu
from jax.experimental.pallas import tpu_sc as plsc

NC, NS, LANES = 2, 16, 16
NW = NC * NS
CHUNK = 80


def _node_mm_body(x_ref, w1_ref, b1_ref, a_ref, b_ref):
    xb = x_ref[...]
    d = x_ref.shape[1]
    w1a = w1_ref[:d, :]
    w1b = w1_ref[d:, :]
    a_ref[...] = jnp.dot(xb, w1a - w1b, preferred_element_type=jnp.float32) + b1_ref[...]
    b_ref[...] = jnp.dot(xb, w1b, preferred_element_type=jnp.float32)


def _node_mm(x, W1, b1, block_n=2000):
    n, d = x.shape
    grid = n // block_n
    return pl.pallas_call(
        _node_mm_body,
        grid=(grid,),
        in_specs=[
            pl.BlockSpec((block_n, d), lambda i: (i, 0)),
            pl.BlockSpec((2 * d, d), lambda i: (0, 0)),
            pl.BlockSpec((1, d), lambda i: (0, 0)),
        ],
        out_specs=[
            pl.BlockSpec((block_n, d), lambda i: (i, 0)),
            pl.BlockSpec((block_n, d), lambda i: (i, 0)),
        ],
        out_shape=[
            jax.ShapeDtypeStruct((n, d), jnp.float32),
            jax.ShapeDtypeStruct((n, d), jnp.float32),
        ],
    )(x, W1, b1)


def _gather_body(CHUNK, a_hbm, b_hbm, dst_hbm, src_hbm, h1_hbm,
                 dsti, srci, arows, brows, hrows,
                 gsem0, gsem1, gsem2, wsem0, wsem1, wsem2):
    e, d = h1_hbm.shape
    per_w = e // NW
    n_chunks = per_w // CHUNK
    assert n_chunks % 3 == 2
    triples = (n_chunks - 2) // 3
    c_ax = lax.axis_index("c")
    s_ax = lax.axis_index("s")
    wid = s_ax * NC + c_ax
    base_w = wid * per_w
    gsems = (gsem0, gsem1, gsem2)
    wsems = (wsem0, wsem1, wsem2)

    pltpu.sync_copy(dst_hbm.at[pl.ds(base_w, per_w)], dsti)
    pltpu.sync_copy(src_hbm.at[pl.ds(base_w, per_w)], srci)

    def issue_gather(ci, b):
        i0 = ci * CHUNK
        pltpu.async_copy(a_hbm.at[dsti.at[pl.ds(i0, CHUNK)]], arows.at[b], gsems[b])
        pltpu.async_copy(b_hbm.at[srci.at[pl.ds(i0, CHUNK)]], brows.at[b], gsems[b])

    def wait_gather(b):
        pltpu.make_async_copy(a_hbm.at[pl.ds(0, CHUNK)], arows.at[b], gsems[b]).wait()
        pltpu.make_async_copy(b_hbm.at[pl.ds(0, CHUNK)], brows.at[b], gsems[b]).wait()

    def wait_wb(b):
        pltpu.make_async_copy(hrows.at[b], h1_hbm.at[pl.ds(0, CHUNK)], wsems[b]).wait()

    def compute(b):
        ar = arows.at[b]
        br = brows.at[b]
        hr = hrows.at[b]

        def row_body(r, rc):
            r8 = r * 8
            for rr in range(8):
                for g in range(d // LANES):
                    sl = pl.ds(g * LANES, LANES)
                    v = ar[r8 + rr, sl] + br[r8 + rr, sl]
                    hr[r8 + rr, sl] = jnp.maximum(v, 0.01 * v)
            return rc

        lax.fori_loop(0, CHUNK // 8, row_body, 0)

    for b in range(3):
        issue_gather(b, b)

    def triple_body(q, carry):
        for i in range(3):
            v = q * 3 + i
            wait_gather(i)

            @pl.when(q >= 1)
            def _():
                wait_wb(i)

            compute(i)
            pltpu.async_copy(hrows.at[i], h1_hbm.at[pl.ds(base_w + v * CHUNK, CHUNK)],
                             wsems[i])
            if i < 2:
                issue_gather(v + 3, i)
            else:
                @pl.when(q < triples - 1)
                def _():
                    issue_gather(v + 3, i)
        return carry

    lax.fori_loop(0, triples, triple_body, 0)

    wait_gather(0)
    wait_wb(0)
    compute(0)
    pltpu.async_copy(hrows.at[0], h1_hbm.at[pl.ds(base_w + (n_chunks - 2) * CHUNK, CHUNK)],
                     wsems[0])
    wait_gather(1)
    wait_wb(1)
    compute(1)
    pltpu.sync_copy(hrows.at[1], h1_hbm.at[pl.ds(base_w + (n_chunks - 1) * CHUNK, CHUNK)])
    wait_wb(2)
    wait_wb(0)


def _edge_gather(A, B, dst, src, E, chunk=CHUNK):
    d = A.shape[1]
    per_w = E // NW
    mesh = plsc.VectorSubcoreMesh(core_axis_name="c", subcore_axis_name="s")
    return pl.kernel(
        functools.partial(_gather_body, chunk),
        out_type=jax.ShapeDtypeStruct((E, d), jnp.float32),
        mesh=mesh,
        scratch_types=[
            pltpu.VMEM((per_w,), jnp.int32),
            pltpu.VMEM((per_w,), jnp.int32),
            pltpu.VMEM((3, chunk, d), jnp.float32),
            pltpu.VMEM((3, chunk, d), jnp.float32),
            pltpu.VMEM((3, chunk, d), jnp.float32),
            pltpu.SemaphoreType.DMA,
            pltpu.SemaphoreType.DMA,
            pltpu.SemaphoreType.DMA,
            pltpu.SemaphoreType.DMA,
            pltpu.SemaphoreType.DMA,
            pltpu.SemaphoreType.DMA,
        ],
    )(A, B, dst, src)


def _edge_mm_body(h1_ref, w2_ref, b2_ref, h2_ref):
    h = jnp.dot(h1_ref[...], w2_ref[...], preferred_element_type=jnp.float32) + b2_ref[...]
    h2_ref[...] = jnp.maximum(h, 0.01 * h)


def _edge_mm(h1, W2, b2, block_e=16000):
    e, d = h1.shape
    grid = e // block_e
    return pl.pallas_call(
        _edge_mm_body,
        grid=(grid,),
        in_specs=[
            pl.BlockSpec((block_e, d), lambda i: (i, 0)),
            pl.BlockSpec((d, d), lambda i: (0, 0)),
            pl.BlockSpec((1, d), lambda i: (0, 0)),
        ],
        out_specs=pl.BlockSpec((block_e, d), lambda i: (i, 0)),
        out_shape=jax.ShapeDtypeStruct((e, d), jnp.float32),
    )(h1, W2, b2)


def _scatter_body(CHUNK, h2_hbm, dst_hbm, zeros_hbm, parts_hbm,
                  dstv0, dstv1, dstv2, dstv3, rows, acc,
                  lsem0, lsem1, lsem2, lsem3,
                  ssem0, ssem1, ssem2, ssem3):
    e, d = h2_hbm.shape
    n = zeros_hbm.shape[0]
    per_w = e // NW
    n_chunks = per_w // CHUNK
    assert n_chunks % 4 == 1
    quads = n_chunks // 4
    stripe = (n // NS) // 8 * 8
    tail = n - NS * stripe
    c_ax = lax.axis_index("c")
    s_ax = lax.axis_index("s")
    wid = s_ax * NC + c_ax
    base_w = wid * per_w
    dstvs = (dstv0, dstv1, dstv2, dstv3)
    lsems = (lsem0, lsem1, lsem2, lsem3)
    ssems = (ssem0, ssem1, ssem2, ssem3)

    pltpu.sync_copy(zeros_hbm.at[pl.ds(s_ax * stripe, stripe)],
                    acc.at[pl.ds(s_ax * stripe, stripe)])
    if tail:
        @pl.when(s_ax == 0)
        def _():
            pltpu.sync_copy(zeros_hbm.at[pl.ds(NS * stripe, tail)],
                            acc.at[pl.ds(NS * stripe, tail)])
    plsc.subcore_barrier()

    def issue_load(ci, b):
        base = base_w + ci * CHUNK
        pltpu.async_copy(dst_hbm.at[pl.ds(base, CHUNK)], dstvs[b], lsems[b])
        pltpu.async_copy(h2_hbm.at[pl.ds(base, CHUNK)], rows.at[b], lsems[b])

    def wait_load(b):
        pltpu.make_async_copy(dst_hbm.at[pl.ds(0, CHUNK)], dstvs[b], lsems[b]).wait()
        pltpu.make_async_copy(h2_hbm.at[pl.ds(0, CHUNK)], rows.at[b], lsems[b]).wait()

    def scatter_go(b):
        pltpu.async_copy(rows.at[b], acc.at[dstvs[b]], ssems[b], add=True)

    def wait_scatter(b):
        pltpu.make_async_copy(h2_hbm.at[pl.ds(0, CHUNK)], rows.at[b], ssems[b]).wait()

    issue_load(0, 0)
    issue_load(1, 1)

    def quad_body(q, carry):
        for i in range(4):
            v = q * 4 + i
            wait_load(i)
            scatter_go(i)
            bw = (i + 2) % 4
            if i < 2:
                @pl.when(q >= 1)
                def _():
                    wait_scatter(bw)

                issue_load(v + 2, bw)
            else:
                @pl.when(v + 2 < n_chunks)
                def _():
                    wait_scatter(bw)
                    issue_load(v + 2, bw)
        return carry

    lax.fori_loop(0, quads, quad_body, 0)
    wait_load(0)
    scatter_go(0)
    for b in range(4):
        wait_scatter(b)

    plsc.subcore_barrier()
    pltpu.sync_copy(acc.at[pl.ds(s_ax * stripe, stripe)],
                    parts_hbm.at[c_ax, pl.ds(s_ax * stripe, stripe)])
    if tail:
        @pl.when(s_ax == 0)
        def _():
            pltpu.sync_copy(acc.at[pl.ds(NS * stripe, tail)],
                            parts_hbm.at[c_ax, pl.ds(NS * stripe, tail)])


def _edge_scatter(h2, dst, zeros, N, chunk=CHUNK):
    e, d = h2.shape
    mesh = plsc.VectorSubcoreMesh(core_axis_name="c", subcore_axis_name="s")
    return pl.kernel(
        functools.partial(_scatter_body, chunk),
        out_type=jax.ShapeDtypeStruct((NC, N, d), jnp.float32),
        mesh=mesh,
        scratch_types=[
            pltpu.VMEM((chunk,), jnp.int32),
            pltpu.VMEM((chunk,), jnp.int32),
            pltpu.VMEM((chunk,), jnp.int32),
            pltpu.VMEM((chunk,), jnp.int32),
            pltpu.VMEM((4, chunk, d), jnp.float32),
            pltpu.VMEM_SHARED((N, d), jnp.float32),
            pltpu.SemaphoreType.DMA,
            pltpu.SemaphoreType.DMA,
            pltpu.SemaphoreType.DMA,
            pltpu.SemaphoreType.DMA,
            pltpu.SemaphoreType.DMA,
            pltpu.SemaphoreType.DMA,
            pltpu.SemaphoreType.DMA,
            pltpu.SemaphoreType.DMA,
        ],
    )(h2, dst, zeros)


def _combine_body(*refs):
    o_ref = refs[-1]
    acc = refs[0][...]
    for r in refs[1:-1]:
        acc = acc + r[...]
    o_ref[...] = acc


def _combine(parts_list, block_n=2000):
    terms = [p[i] for p in parts_list for i in range(p.shape[0])]
    n, d = terms[0].shape
    grid = n // block_n
    return pl.pallas_call(
        _combine_body,
        grid=(grid,),
        in_specs=[pl.BlockSpec((block_n, d), lambda i: (i, 0)) for _ in terms],
        out_specs=pl.BlockSpec((block_n, d), lambda i: (i, 0)),
        out_shape=jax.ShapeDtypeStruct((n, d), jnp.float32),
    )(*terms)


def kernel(x, edge_index, W1, b1, W2, b2):
    n, d = x.shape
    e = edge_index.shape[1]
    src = edge_index[0]
    dst = edge_index[1]
    A, B = _node_mm(x, W1, b1.reshape(1, d))
    zeros = jnp.zeros((n, d), jnp.float32)
    h1 = _edge_gather(A, B, dst, src, e)
    h2 = _edge_mm(h1, W2, b2.reshape(1, d))
    parts = _edge_scatter(h2, dst, zeros, n)
    return _combine([parts])
